# Initial kernel scaffold; baseline (speedup 1.0000x reference)
import jax
import jax.numpy as jnp
from jax.experimental import pallas as pl


def kernel(x, edge_index, W1, b1, Wmu, bmu, Wls, bls, eps):
    raise NotImplementedError("write your pallas kernel here")


# speedup vs baseline: 17.1142x; 17.1142x over previous
import functools

import jax
import jax.numpy as jnp
from jax import lax
from jax.experimental import pallas as pl
from jax.experimental.pallas import tpu as pltpu
from jax.experimental.pallas import tpu_sc as plsc

_NC = 2
_NS = 16
_NW = _NC * _NS

_MESH = dict(core_axis_name="c", subcore_axis_name="s")


def _chunk_of(ept):
    for c in range(128, 0, -8):
        if ept % c == 0:
            return c
    raise ValueError(f"no aligned chunk divides {ept}")


def _row_split(n):
    rpt = (-(-n // _NS) + 7) // 8 * 8
    last = n - (_NS - 1) * rpt
    assert last > 0
    return rpt, last


def _sc_degree(dst, n):
    e = dst.shape[0]
    assert e % _NW == 0
    ept = e // _NW
    c = _chunk_of(ept)
    nsteps = ept // c
    rpt, last = _row_split(n)

    @functools.partial(
        pl.kernel,
        out_type=(jax.ShapeDtypeStruct((n, 16), jnp.float32),
                  jax.ShapeDtypeStruct((n, 16), jnp.float32)),
        mesh=plsc.VectorSubcoreMesh(**_MESH),
        compiler_params=pltpu.CompilerParams(use_tc_tiling_on_sc=False),
        scratch_types=[
            pltpu.VMEM((c,), jnp.int32),
            pltpu.VMEM((c, 16), jnp.float32),
            pltpu.VMEM((rpt, 16), jnp.float32),
            pltpu.VMEM_SHARED((n, 16), jnp.float32),
        ],
    )
    def k(dst_hbm, out0, out1, idx_v, ones_v, zbuf_v, acc_sh):
        cid = lax.axis_index("c")
        sid = lax.axis_index("s")
        wid = cid * _NS + sid

        def fill_ones(i, carry):
            ones_v[i, :] = jnp.ones((16,), jnp.float32)
            return carry
        lax.fori_loop(0, c, fill_ones, 0)

        def fill_zero(i, carry):
            zbuf_v[i, :] = jnp.zeros((16,), jnp.float32)
            return carry
        lax.fori_loop(0, rpt, fill_zero, 0)

        row0 = sid * rpt

        @pl.when(sid < _NS - 1)
        def _():
            pltpu.sync_copy(zbuf_v, acc_sh.at[pl.ds(row0, rpt)])

        @pl.when(sid == _NS - 1)
        def _():
            pltpu.sync_copy(zbuf_v.at[pl.ds(0, last)], acc_sh.at[pl.ds(row0, last)])

        plsc.subcore_barrier()

        base = wid * ept

        def step(j, carry):
            pltpu.sync_copy(dst_hbm.at[pl.ds(base + j * c, c)], idx_v)
            pltpu.sync_copy(ones_v, acc_sh.at[idx_v], add=True)
            return carry
        lax.fori_loop(0, nsteps, step, 0)

        plsc.subcore_barrier()

        @pl.when(sid < _NS - 1)
        def _():
            @pl.when(cid == 0)
            def _():
                pltpu.sync_copy(acc_sh.at[pl.ds(row0, rpt)], out0.at[pl.ds(row0, rpt)])

            @pl.when(cid == 1)
            def _():
                pltpu.sync_copy(acc_sh.at[pl.ds(row0, rpt)], out1.at[pl.ds(row0, rpt)])

        @pl.when(sid == _NS - 1)
        def _():
            @pl.when(cid == 0)
            def _():
                pltpu.sync_copy(acc_sh.at[pl.ds(row0, last)], out0.at[pl.ds(row0, last)])

            @pl.when(cid == 1)
            def _():
                pltpu.sync_copy(acc_sh.at[pl.ds(row0, last)], out1.at[pl.ds(row0, last)])

    return k(dst)


def _sc_scatter(src, dst, table):
    n, d = table.shape
    e = src.shape[0]
    assert e % _NW == 0 and d % 16 == 0
    ept = e // _NW
    c = _chunk_of(ept)
    nsteps = ept // c
    rpt, last = _row_split(n)

    @functools.partial(
        pl.kernel,
        out_type=(jax.ShapeDtypeStruct((n, d), jnp.float32),
                  jax.ShapeDtypeStruct((n, d), jnp.float32)),
        mesh=plsc.VectorSubcoreMesh(**_MESH),
        compiler_params=pltpu.CompilerParams(use_tc_tiling_on_sc=False),
        scratch_types=[
            pltpu.VMEM((c,), jnp.int32),
            pltpu.VMEM((c,), jnp.int32),
            pltpu.VMEM((c, d), jnp.float32),
            pltpu.VMEM((rpt, d), jnp.float32),
            pltpu.VMEM_SHARED((n, d), jnp.float32),
            pltpu.SemaphoreType.DMA,
        ],
    )
    def k(src_hbm, dst_hbm, tbl_hbm, out0, out1,
          sidx_v, didx_v, rows_v, zbuf_v, acc_sh, sem):
        cid = lax.axis_index("c")
        sid = lax.axis_index("s")
        wid = cid * _NS + sid

        def fill_zero(i, carry):
            for q in range(d // 16):
                zbuf_v[i, pl.ds(q * 16, 16)] = jnp.zeros((16,), jnp.float32)
            return carry
        lax.fori_loop(0, rpt, fill_zero, 0)

        row0 = sid * rpt

        @pl.when(sid < _NS - 1)
        def _():
            pltpu.sync_copy(zbuf_v, acc_sh.at[pl.ds(row0, rpt)])

        @pl.when(sid == _NS - 1)
        def _():
            pltpu.sync_copy(zbuf_v.at[pl.ds(0, last)], acc_sh.at[pl.ds(row0, last)])

        plsc.subcore_barrier()

        base = wid * ept

        def step(j, carry):
            off = base + j * c
            pltpu.sync_copy(src_hbm.at[pl.ds(off, c)], sidx_v)
            pltpu.sync_copy(dst_hbm.at[pl.ds(off, c)], didx_v)
            pltpu.async_copy(tbl_hbm.at[sidx_v], rows_v, sem).wait()
            pltpu.sync_copy(rows_v, acc_sh.at[didx_v], add=True)
            return carry
        lax.fori_loop(0, nsteps, step, 0)

        plsc.subcore_barrier()

        @pl.when(sid < _NS - 1)
        def _():
            @pl.when(cid == 0)
            def _():
                pltpu.sync_copy(acc_sh.at[pl.ds(row0, rpt)], out0.at[pl.ds(row0, rpt)])

            @pl.when(cid == 1)
            def _():
                pltpu.sync_copy(acc_sh.at[pl.ds(row0, rpt)], out1.at[pl.ds(row0, rpt)])

        @pl.when(sid == _NS - 1)
        def _():
            @pl.when(cid == 0)
            def _():
                pltpu.sync_copy(acc_sh.at[pl.ds(row0, last)], out0.at[pl.ds(row0, last)])

            @pl.when(cid == 1)
            def _():
                pltpu.sync_copy(acc_sh.at[pl.ds(row0, last)], out1.at[pl.ds(row0, last)])

    return k(src, dst, table)


def _tc_prep(x, w1, d0, d1):
    n = x.shape[0]
    dh = w1.shape[1]

    def body(x_ref, w_ref, d0_ref, d1_ref, g_ref, dinv_ref):
        deg = d0_ref[...][:, 0:1] + d1_ref[...][:, 0:1] + 1.0
        dinv = lax.rsqrt(deg)
        h = jnp.dot(x_ref[...], w_ref[...], preferred_element_type=jnp.float32)
        g_ref[...] = h * dinv
        dinv_ref[...] = dinv

    return pl.pallas_call(
        body,
        out_shape=(jax.ShapeDtypeStruct((n, dh), jnp.float32),
                   jax.ShapeDtypeStruct((n, 1), jnp.float32)),
    )(x, w1, d0, d1)


def _tc_hidden(p0, p1, g1, dinv, b1):
    def body(p0_ref, p1_ref, g1_ref, dinv_ref, b_ref, g2_ref):
        dinv = dinv_ref[...]
        agg = (p0_ref[...] + p1_ref[...] + g1_ref[...]) * dinv
        hidden = jnp.maximum(agg + b_ref[...], 0.0)
        g2_ref[...] = hidden * dinv

    return pl.pallas_call(
        body,
        out_shape=jax.ShapeDtypeStruct(g1.shape, jnp.float32),
    )(p0, p1, g1, dinv, b1)


def _tc_heads(q0, q1, g2, dinv, wmu, bmu, wls, bls, eps):
    n = g2.shape[0]
    dz = wmu.shape[1]

    def body(q0_ref, q1_ref, g2_ref, dinv_ref, wmu_ref, bmu_ref,
             wls_ref, bls_ref, eps_ref, z_ref, mu_ref, ls_ref):
        a2 = (q0_ref[...] + q1_ref[...] + g2_ref[...]) * dinv_ref[...]
        mu = jnp.dot(a2, wmu_ref[...], preferred_element_type=jnp.float32) + bmu_ref[...]
        ls = jnp.dot(a2, wls_ref[...], preferred_element_type=jnp.float32) + bls_ref[...]
        z_ref[...] = mu + eps_ref[...] * jnp.exp(ls)
        mu_ref[...] = mu
        ls_ref[...] = ls

    shape = jax.ShapeDtypeStruct((n, dz), jnp.float32)
    return pl.pallas_call(
        body,
        out_shape=(shape, shape, shape),
    )(q0, q1, g2, dinv, wmu, bmu, wls, bls, eps)


def kernel(x, edge_index, W1, b1, Wmu, bmu, Wls, bls, eps):
    n = x.shape[0]
    src = edge_index[0]
    dst = edge_index[1]

    deg0, deg1 = _sc_degree(dst, n)
    g1, dinv = _tc_prep(x, W1, deg0, deg1)
    p0, p1 = _sc_scatter(src, dst, g1)
    g2 = _tc_hidden(p0, p1, g1, dinv, b1.reshape(1, -1))
    q0, q1 = _sc_scatter(src, dst, g2)
    z, mu, logstd = _tc_heads(q0, q1, g2, dinv, Wmu, bmu.reshape(1, -1),
                              Wls, bls.reshape(1, -1), eps)
    return (z, mu, logstd)


# speedup vs baseline: 49.4024x; 2.8866x over previous
import functools

import jax
import jax.numpy as jnp
from jax import lax
from jax.experimental import pallas as pl
from jax.experimental.pallas import tpu as pltpu
from jax.experimental.pallas import tpu_sc as plsc

_NC = 2
_NS = 16
_NW = _NC * _NS

_MESH = dict(core_axis_name="c", subcore_axis_name="s")
_ZROWS = 40
_NBUF = 12


def _chunk_of(ept):
    for c in range(128, 0, -8):
        if ept % c == 0:
            return c
    raise ValueError(f"no aligned chunk divides {ept}")


def _row_split(n):
    rpt = (-(-n // _NS) + 7) // 8 * 8
    last = n - (_NS - 1) * rpt
    assert last > 0
    return rpt, last


def _edge_grid(e):
    assert e % _NW == 0
    ept = e // _NW
    c = _chunk_of(ept)
    return ept, c, ept // c


def _sc_degree(dst3, n):
    _, nsteps, c = dst3.shape

    @functools.partial(
        pl.kernel,
        out_type=(jax.ShapeDtypeStruct((n, 16), jnp.float32),
                  jax.ShapeDtypeStruct((n, 16), jnp.float32)),
        mesh=plsc.VectorSubcoreMesh(**_MESH),
        compiler_params=pltpu.CompilerParams(use_tc_tiling_on_sc=False),
        scratch_types=[
            pltpu.VMEM((nsteps, c), jnp.int32),
            pltpu.VMEM((c, 16), jnp.float32),
            pltpu.VMEM((_row_split(n)[0], 16), jnp.float32),
            pltpu.VMEM_SHARED((n, 16), jnp.float32),
            pltpu.SemaphoreType.DMA,
        ],
    )
    def k(dst_hbm, out0, out1, idx_v, ones_v, zbuf_v, acc_sh, sem):
        cid = lax.axis_index("c")
        sid = lax.axis_index("s")
        wid = cid * _NS + sid
        rpt, last = _row_split(n)

        def fill_ones(i, carry):
            ones_v[i, :] = jnp.ones((16,), jnp.float32)
            return carry
        lax.fori_loop(0, c, fill_ones, 0)

        def fill_zero(i, carry):
            zbuf_v[i, :] = jnp.zeros((16,), jnp.float32)
            return carry
        lax.fori_loop(0, rpt, fill_zero, 0)

        row0 = sid * rpt

        @pl.when(sid < _NS - 1)
        def _():
            pltpu.sync_copy(zbuf_v, acc_sh.at[pl.ds(row0, rpt)])

        @pl.when(sid == _NS - 1)
        def _():
            pltpu.sync_copy(zbuf_v.at[pl.ds(0, last)], acc_sh.at[pl.ds(row0, last)])

        pltpu.sync_copy(dst_hbm.at[wid], idx_v)
        plsc.subcore_barrier()

        def fire(j, carry):
            pltpu.async_copy(ones_v, acc_sh.at[idx_v.at[j]], sem, add=True)
            return carry
        lax.fori_loop(0, nsteps, fire, 0)

        def drain(j, carry):
            pltpu.make_async_copy(ones_v, acc_sh.at[pl.ds(0, c)], sem).wait()
            return carry
        lax.fori_loop(0, nsteps, drain, 0)

        plsc.subcore_barrier()

        @pl.when(sid < _NS - 1)
        def _():
            @pl.when(cid == 0)
            def _():
                pltpu.sync_copy(acc_sh.at[pl.ds(row0, rpt)], out0.at[pl.ds(row0, rpt)])

            @pl.when(cid == 1)
            def _():
                pltpu.sync_copy(acc_sh.at[pl.ds(row0, rpt)], out1.at[pl.ds(row0, rpt)])

        @pl.when(sid == _NS - 1)
        def _():
            @pl.when(cid == 0)
            def _():
                pltpu.sync_copy(acc_sh.at[pl.ds(row0, last)], out0.at[pl.ds(row0, last)])

            @pl.when(cid == 1)
            def _():
                pltpu.sync_copy(acc_sh.at[pl.ds(row0, last)], out1.at[pl.ds(row0, last)])

    return k(dst3)


def _sc_scatter(src3, dst3, table):
    n, d = table.shape
    _, nsteps, c = src3.shape
    k_ahead = _NBUF // 2
    assert nsteps > _NBUF
    ngroups = nsteps // _NBUF
    rem = nsteps - ngroups * _NBUF
    rpt, last = _row_split(n)

    @functools.partial(
        pl.kernel,
        out_type=(jax.ShapeDtypeStruct((n, d), jnp.float32),
                  jax.ShapeDtypeStruct((n, d), jnp.float32)),
        mesh=plsc.VectorSubcoreMesh(**_MESH),
        compiler_params=pltpu.CompilerParams(use_tc_tiling_on_sc=False),
        scratch_types=[
            pltpu.VMEM((nsteps, c), jnp.int32),
            pltpu.VMEM((nsteps, c), jnp.int32),
            pltpu.VMEM((_NBUF, c, d), jnp.float32),
            pltpu.VMEM((_ZROWS, d), jnp.float32),
            pltpu.VMEM_SHARED((n, d), jnp.float32),
            pltpu.SemaphoreType.DMA((_NBUF,)),
            pltpu.SemaphoreType.DMA((_NBUF,)),
            pltpu.SemaphoreType.DMA,
        ],
    )
    def k(src_hbm, dst_hbm, tbl_hbm, out0, out1,
          sidx_v, didx_v, rows_v, zbuf_v, acc_sh, gsem, ssem, zsem):
        cid = lax.axis_index("c")
        sid = lax.axis_index("s")
        wid = cid * _NS + sid

        def fill_zero(i, carry):
            for q in range(d // 16):
                zbuf_v[i, pl.ds(q * 16, 16)] = jnp.zeros((16,), jnp.float32)
            return carry
        lax.fori_loop(0, _ZROWS, fill_zero, 0)

        row0 = sid * rpt

        def zinit(nrows):
            nfull, zrem = divmod(nrows, _ZROWS)

            def zf(t, carry):
                pltpu.async_copy(zbuf_v, acc_sh.at[pl.ds(row0 + t * _ZROWS, _ZROWS)], zsem)
                return carry
            lax.fori_loop(0, nfull, zf, 0)
            if zrem:
                pltpu.async_copy(zbuf_v.at[pl.ds(0, zrem)],
                                 acc_sh.at[pl.ds(row0 + nfull * _ZROWS, zrem)], zsem)

            def zd(t, carry):
                pltpu.make_async_copy(zbuf_v, acc_sh.at[pl.ds(0, _ZROWS)], zsem).wait()
                return carry
            lax.fori_loop(0, nfull, zd, 0)
            if zrem:
                pltpu.make_async_copy(zbuf_v.at[pl.ds(0, zrem)],
                                      acc_sh.at[pl.ds(0, zrem)], zsem).wait()

        @pl.when(sid < _NS - 1)
        def _():
            zinit(rpt)

        @pl.when(sid == _NS - 1)
        def _():
            zinit(last)

        pltpu.sync_copy(src_hbm.at[wid], sidx_v)
        pltpu.sync_copy(dst_hbm.at[wid], didx_v)
        plsc.subcore_barrier()

        def fire_gather(j, b):
            pltpu.async_copy(tbl_hbm.at[sidx_v.at[j]], rows_v.at[b], gsem.at[b])

        def wait_gather(b):
            pltpu.make_async_copy(tbl_hbm.at[pl.ds(0, c)], rows_v.at[b],
                                  gsem.at[b]).wait()

        def fire_scatter(j, b):
            pltpu.async_copy(rows_v.at[b], acc_sh.at[didx_v.at[j]], ssem.at[b],
                             add=True)

        def wait_scatter(b):
            pltpu.make_async_copy(tbl_hbm.at[pl.ds(0, c)], rows_v.at[b],
                                  ssem.at[b]).wait()

        for b in range(k_ahead):
            fire_gather(b, b)

        def step(j, b):
            b2 = (b + k_ahead) % _NBUF
            wait_gather(b)
            fire_scatter(j, b)

            @pl.when(j + k_ahead < nsteps)
            def _():
                @pl.when(j >= k_ahead)
                def _():
                    wait_scatter(b2)
                fire_gather(j + k_ahead, b2)

        def group(i, carry):
            for b in range(_NBUF):
                step(i * _NBUF + b, b)
            return carry
        lax.fori_loop(0, ngroups, group, 0)

        for b in range(rem):
            step(ngroups * _NBUF + b, b)

        for b in range(_NBUF):
            wait_scatter(b)

        plsc.subcore_barrier()

        @pl.when(sid < _NS - 1)
        def _():
            @pl.when(cid == 0)
            def _():
                pltpu.sync_copy(acc_sh.at[pl.ds(row0, rpt)], out0.at[pl.ds(row0, rpt)])

            @pl.when(cid == 1)
            def _():
                pltpu.sync_copy(acc_sh.at[pl.ds(row0, rpt)], out1.at[pl.ds(row0, rpt)])

        @pl.when(sid == _NS - 1)
        def _():
            @pl.when(cid == 0)
            def _():
                pltpu.sync_copy(acc_sh.at[pl.ds(row0, last)], out0.at[pl.ds(row0, last)])

            @pl.when(cid == 1)
            def _():
                pltpu.sync_copy(acc_sh.at[pl.ds(row0, last)], out1.at[pl.ds(row0, last)])

    return k(src3, dst3, table)


def _tc_matmul(x, w1):
    n = x.shape[0]
    dh = w1.shape[1]

    def body(x_ref, w_ref, h_ref):
        h_ref[...] = jnp.dot(x_ref[...], w_ref[...],
                             preferred_element_type=jnp.float32)

    return pl.pallas_call(
        body,
        out_shape=jax.ShapeDtypeStruct((n, dh), jnp.float32),
    )(x, w1)


def _tc_scale(h1, d0, d1):
    n, dh = h1.shape

    def body(h_ref, d0_ref, d1_ref, g_ref, dinv_ref):
        deg = d0_ref[...][:, 0:1] + d1_ref[...][:, 0:1] + 1.0
        dinv = lax.rsqrt(deg)
        g_ref[...] = h_ref[...] * dinv
        dinv_ref[...] = dinv

    return pl.pallas_call(
        body,
        out_shape=(jax.ShapeDtypeStruct((n, dh), jnp.float32),
                   jax.ShapeDtypeStruct((n, 1), jnp.float32)),
    )(h1, d0, d1)


def _tc_hidden(p0, p1, g1, dinv, b1):
    def body(p0_ref, p1_ref, g1_ref, dinv_ref, b_ref, g2_ref):
        dinv = dinv_ref[...]
        agg = (p0_ref[...] + p1_ref[...] + g1_ref[...]) * dinv
        hidden = jnp.maximum(agg + b_ref[...], 0.0)
        g2_ref[...] = hidden * dinv

    return pl.pallas_call(
        body,
        out_shape=jax.ShapeDtypeStruct(g1.shape, jnp.float32),
    )(p0, p1, g1, dinv, b1)


def _tc_heads(q0, q1, g2, dinv, wmu, bmu, wls, bls, eps):
    n = g2.shape[0]
    dz = wmu.shape[1]

    def body(q0_ref, q1_ref, g2_ref, dinv_ref, wmu_ref, bmu_ref,
             wls_ref, bls_ref, eps_ref, z_ref, mu_ref, ls_ref):
        a2 = (q0_ref[...] + q1_ref[...] + g2_ref[...]) * dinv_ref[...]
        mu = jnp.dot(a2, wmu_ref[...], preferred_element_type=jnp.float32) + bmu_ref[...]
        ls = jnp.dot(a2, wls_ref[...], preferred_element_type=jnp.float32) + bls_ref[...]
        z_ref[...] = mu + eps_ref[...] * jnp.exp(ls)
        mu_ref[...] = mu
        ls_ref[...] = ls

    shape = jax.ShapeDtypeStruct((n, dz), jnp.float32)
    return pl.pallas_call(
        body,
        out_shape=(shape, shape, shape),
    )(q0, q1, g2, dinv, wmu, bmu, wls, bls, eps)


def kernel(x, edge_index, W1, b1, Wmu, bmu, Wls, bls, eps):
    n = x.shape[0]
    e = edge_index.shape[1]
    _, c, nsteps = _edge_grid(e)
    src3 = edge_index[0].reshape(_NW, nsteps, c)
    dst3 = edge_index[1].reshape(_NW, nsteps, c)

    h1 = _tc_matmul(x, W1)
    deg0, deg1 = _sc_degree(dst3, n)
    g1, dinv = _tc_scale(h1, deg0, deg1)
    p0, p1 = _sc_scatter(src3, dst3, g1)
    g2 = _tc_hidden(p0, p1, g1, dinv, b1.reshape(1, -1))
    q0, q1 = _sc_scatter(src3, dst3, g2)
    z, mu, logstd = _tc_heads(q0, q1, g2, dinv, Wmu, bmu.reshape(1, -1),
                              Wls, bls.reshape(1, -1), eps)
    return (z, mu, logstd)
